# initial kernel scaffold (unmeasured)
import jax
import jax.numpy as jnp
from jax import lax
from jax.experimental import pallas as pl
from jax.experimental.pallas import tpu as pltpu


def kernel(
    x,
):
    def body(*refs):
        pass

    out_shape = jax.ShapeDtypeStruct(..., jnp.float32)
    return pl.pallas_call(body, out_shape=out_shape)(...)


# baseline (device time: 130304 ns/iter reference)
import jax
import jax.numpy as jnp
from jax import lax
from jax.experimental import pallas as pl
from jax.experimental.pallas import tpu as pltpu

K = 32
BM = 128


def _topk_desc(xv, k):
    outs = []
    for _ in range(k):
        m = jnp.max(xv, axis=1, keepdims=True)
        outs.append(m)
        xv = jnp.where(xv == m, -jnp.inf, xv)
    return jnp.concatenate(outs, axis=1)


def _local_topk_body(x_ref, out_ref):
    out_ref[...] = _topk_desc(x_ref[...], K)


def _merge_body(loc_ref, out_ref, comm_ref, send_sem, recv_sem):
    my_x = lax.axis_index("x")
    my_y = lax.axis_index("y")
    peer = (1 - my_x, my_y)

    barrier_sem = pltpu.get_barrier_semaphore()
    pl.semaphore_signal(
        barrier_sem, inc=1, device_id=peer, device_id_type=pl.DeviceIdType.MESH
    )
    pl.semaphore_wait(barrier_sem, 1)

    rdma = pltpu.make_async_remote_copy(
        src_ref=loc_ref,
        dst_ref=comm_ref,
        send_sem=send_sem,
        recv_sem=recv_sem,
        device_id=peer,
        device_id_type=pl.DeviceIdType.MESH,
    )
    rdma.start()
    rdma.wait()

    cat = jnp.concatenate([loc_ref[...], comm_ref[...]], axis=1)
    out_ref[...] = _topk_desc(cat, K)


def kernel(x):
    m, n_loc = x.shape

    loc = pl.pallas_call(
        _local_topk_body,
        grid=(m // BM,),
        in_specs=[pl.BlockSpec((BM, n_loc), lambda i: (i, 0))],
        out_specs=pl.BlockSpec((BM, K), lambda i: (i, 0)),
        out_shape=jax.ShapeDtypeStruct((m, K), jnp.float32),
    )(x)

    return pl.pallas_call(
        _merge_body,
        out_shape=jax.ShapeDtypeStruct((m, K), jnp.float32),
        in_specs=[pl.BlockSpec(memory_space=pltpu.VMEM)],
        out_specs=pl.BlockSpec(memory_space=pltpu.VMEM),
        scratch_shapes=[
            pltpu.VMEM((m, K), jnp.float32),
            pltpu.SemaphoreType.DMA,
            pltpu.SemaphoreType.DMA,
        ],
        compiler_params=pltpu.CompilerParams(collective_id=0),
    )(loc)


# device time: 72894 ns/iter; 1.7876x vs baseline; 1.7876x over previous
import jax
import jax.numpy as jnp
from jax import lax
from jax.experimental import pallas as pl
from jax.experimental.pallas import tpu as pltpu

K = 32
BM = 128


def _topk_desc(xv, k):
    m = jnp.max(xv, axis=1, keepdims=True)
    outs = [m]
    for _ in range(k - 1):
        m = jnp.max(jnp.where(xv < m, xv, -jnp.inf), axis=1, keepdims=True)
        outs.append(m)
    return jnp.concatenate(outs, axis=1)


def _local_topk_body(y_ref, x_ref, out_ref):
    del y_ref
    out_ref[...] = _topk_desc(x_ref[...], K)


def _merge_body(loc_ref, out_ref, comm_ref, sx_send, sx_recv, sy_send, sy_recv):
    my_x = lax.axis_index("x")
    my_y = lax.axis_index("y")
    x_peer = (1 - my_x, my_y)
    y_peer = (my_x, 1 - my_y)
    half_m = loc_ref.shape[0]

    barrier_sem = pltpu.get_barrier_semaphore()
    for peer in (x_peer, y_peer):
        pl.semaphore_signal(
            barrier_sem, inc=1, device_id=peer,
            device_id_type=pl.DeviceIdType.MESH,
        )
    pl.semaphore_wait(barrier_sem, 2)

    rdma1 = pltpu.make_async_remote_copy(
        src_ref=loc_ref,
        dst_ref=comm_ref,
        send_sem=sx_send,
        recv_sem=sx_recv,
        device_id=x_peer,
        device_id_type=pl.DeviceIdType.MESH,
    )
    rdma1.start()
    rdma1.wait()

    cat = jnp.concatenate([loc_ref[...], comm_ref[...]], axis=1)
    row0 = my_y * half_m
    out_ref[pl.ds(row0, half_m), :] = _topk_desc(cat, K)

    rdma2 = pltpu.make_async_remote_copy(
        src_ref=out_ref.at[pl.ds(row0, half_m), :],
        dst_ref=out_ref.at[pl.ds(row0, half_m), :],
        send_sem=sy_send,
        recv_sem=sy_recv,
        device_id=y_peer,
        device_id_type=pl.DeviceIdType.MESH,
    )
    rdma2.start()
    rdma2.wait()


def kernel(x):
    m, n_loc = x.shape
    half_m = m // 2
    n_blocks = half_m // BM

    my_y = jnp.full((1,), lax.axis_index("y"), jnp.int32)

    loc = pl.pallas_call(
        _local_topk_body,
        grid_spec=pltpu.PrefetchScalarGridSpec(
            num_scalar_prefetch=1,
            grid=(n_blocks,),
            in_specs=[
                pl.BlockSpec((BM, n_loc), lambda i, y: (y[0] * n_blocks + i, 0))
            ],
            out_specs=pl.BlockSpec((BM, K), lambda i, y: (i, 0)),
        ),
        out_shape=jax.ShapeDtypeStruct((half_m, K), jnp.float32),
    )(my_y, x)

    return pl.pallas_call(
        _merge_body,
        out_shape=jax.ShapeDtypeStruct((m, K), jnp.float32),
        in_specs=[pl.BlockSpec(memory_space=pltpu.VMEM)],
        out_specs=pl.BlockSpec(memory_space=pltpu.VMEM),
        scratch_shapes=[
            pltpu.VMEM((half_m, K), jnp.float32),
            pltpu.SemaphoreType.DMA,
            pltpu.SemaphoreType.DMA,
            pltpu.SemaphoreType.DMA,
            pltpu.SemaphoreType.DMA,
        ],
        compiler_params=pltpu.CompilerParams(collective_id=0),
    )(loc)


# device time: 36043 ns/iter; 3.6152x vs baseline; 2.0224x over previous
import jax
import jax.numpy as jnp
from jax import lax
from jax.experimental import pallas as pl
from jax.experimental.pallas import tpu as pltpu

K = 32
BM = 128


def _topk_desc(xv, k):
    m = jnp.max(xv, axis=1, keepdims=True)
    outs = [m]
    for _ in range(k - 1):
        m = jnp.max(jnp.where(xv < m, xv, -jnp.inf), axis=1, keepdims=True)
        outs.append(m)
    return jnp.concatenate(outs, axis=1)


LANES = 128
PER_LANE = 8


def _local_topk_body(y_ref, x_ref, out_ref):
    del y_ref
    n_tiles = x_ref.shape[1] // LANES
    top = [x_ref[:, t * LANES : (t + 1) * LANES] for t in range(PER_LANE)]
    for j in range(PER_LANE):
        for i in range(PER_LANE - 1 - j):
            hi = jnp.maximum(top[i], top[i + 1])
            lo = jnp.minimum(top[i], top[i + 1])
            top[i], top[i + 1] = hi, lo
    for t in range(PER_LANE, n_tiles):
        v = x_ref[:, t * LANES : (t + 1) * LANES]
        for j in range(PER_LANE):
            hi = jnp.maximum(top[j], v)
            v = jnp.minimum(top[j], v)
            top[j] = hi
    cand = jnp.concatenate(top, axis=1)
    out_ref[...] = _topk_desc(cand, K)


def _merge_body(loc_ref, out_ref, comm_ref, sx_send, sx_recv, sy_send, sy_recv):
    my_x = lax.axis_index("x")
    my_y = lax.axis_index("y")
    x_peer = (1 - my_x, my_y)
    y_peer = (my_x, 1 - my_y)
    half_m = loc_ref.shape[0]

    barrier_sem = pltpu.get_barrier_semaphore()
    for peer in (x_peer, y_peer):
        pl.semaphore_signal(
            barrier_sem, inc=1, device_id=peer,
            device_id_type=pl.DeviceIdType.MESH,
        )
    pl.semaphore_wait(barrier_sem, 2)

    rdma1 = pltpu.make_async_remote_copy(
        src_ref=loc_ref,
        dst_ref=comm_ref,
        send_sem=sx_send,
        recv_sem=sx_recv,
        device_id=x_peer,
        device_id_type=pl.DeviceIdType.MESH,
    )
    rdma1.start()
    rdma1.wait()

    cat = jnp.concatenate([loc_ref[...], comm_ref[...]], axis=1)
    row0 = my_y * half_m
    out_ref[pl.ds(row0, half_m), :] = _topk_desc(cat, K)

    rdma2 = pltpu.make_async_remote_copy(
        src_ref=out_ref.at[pl.ds(row0, half_m), :],
        dst_ref=out_ref.at[pl.ds(row0, half_m), :],
        send_sem=sy_send,
        recv_sem=sy_recv,
        device_id=y_peer,
        device_id_type=pl.DeviceIdType.MESH,
    )
    rdma2.start()
    rdma2.wait()


def kernel(x):
    m, n_loc = x.shape
    half_m = m // 2
    n_blocks = half_m // BM

    my_y = jnp.full((1,), lax.axis_index("y"), jnp.int32)

    loc = pl.pallas_call(
        _local_topk_body,
        grid_spec=pltpu.PrefetchScalarGridSpec(
            num_scalar_prefetch=1,
            grid=(n_blocks,),
            in_specs=[
                pl.BlockSpec((BM, n_loc), lambda i, y: (y[0] * n_blocks + i, 0))
            ],
            out_specs=pl.BlockSpec((BM, K), lambda i, y: (i, 0)),
        ),
        out_shape=jax.ShapeDtypeStruct((half_m, K), jnp.float32),
    )(my_y, x)

    return pl.pallas_call(
        _merge_body,
        out_shape=jax.ShapeDtypeStruct((m, K), jnp.float32),
        in_specs=[pl.BlockSpec(memory_space=pltpu.VMEM)],
        out_specs=pl.BlockSpec(memory_space=pltpu.VMEM),
        scratch_shapes=[
            pltpu.VMEM((half_m, K), jnp.float32),
            pltpu.SemaphoreType.DMA,
            pltpu.SemaphoreType.DMA,
            pltpu.SemaphoreType.DMA,
            pltpu.SemaphoreType.DMA,
        ],
        compiler_params=pltpu.CompilerParams(collective_id=0),
    )(loc)
